# baseline (speedup 1.0000x reference)
import functools

import jax
import jax.numpy as jnp
from jax import lax
from jax.experimental import pallas as pl
from jax.experimental.pallas import tpu as pltpu
from jax.experimental.pallas import tpu_sc as plsc

EPS = 1e-5


def _ln_table_body(table_ref, w_ref, b_ref, out_ref):
    t = table_ref[...]
    mean = jnp.mean(t, axis=-1, keepdims=True)
    var = jnp.mean(jnp.square(t - mean), axis=-1, keepdims=True)
    out_ref[...] = (t - mean) * lax.rsqrt(var + EPS) * w_ref[...] + b_ref[...]


def _normalize_table(table, ln_weight, ln_bias):
    rows, hidden = table.shape
    return pl.pallas_call(
        _ln_table_body,
        out_shape=jax.ShapeDtypeStruct((rows, hidden), jnp.float32),
    )(table, ln_weight.reshape(1, hidden), ln_bias.reshape(1, hidden))


def _make_sc_gather(tokens, hidden, rows):
    info = plsc.get_sparse_core_info()
    nc, ns, nl = info.num_cores, info.num_subcores, info.num_lanes
    nw = nc * ns
    per_w = tokens // nw
    ngroups = per_w // nl
    mesh = plsc.VectorSubcoreMesh(core_axis_name="c", subcore_axis_name="s")

    @functools.partial(
        pl.kernel,
        mesh=mesh,
        out_type=jax.ShapeDtypeStruct((tokens, hidden), jnp.float32),
        scratch_types=[
            pltpu.VMEM((rows, hidden), jnp.float32),
            pltpu.VMEM((2 * rows * rows, hidden), jnp.float32),
            pltpu.VMEM((hidden,), jnp.float32),
            pltpu.VMEM((hidden,), jnp.float32),
            pltpu.VMEM((per_w,), jnp.int32),
            pltpu.VMEM((nl, hidden), jnp.float32),
            pltpu.SemaphoreType.DMA,
        ],
    )
    def sc_gather(table_hbm, w_hbm, b_hbm, idx_hbm, out_hbm,
                  raw_v, tab_v, w_v, b_v, idx_v, drain_v, osem):
        wid = lax.axis_index("s") * nc + lax.axis_index("c")
        base = wid * per_w
        pltpu.sync_copy(table_hbm, raw_v)
        pltpu.sync_copy(w_hbm, w_v)
        pltpu.sync_copy(b_hbm, b_v)
        pltpu.sync_copy(idx_hbm.at[pl.ds(base, per_w)], idx_v)

        unroll = 8
        nch = hidden // nl
        for r in range(rows):

            def stats(i, acc):
                s, s2 = acc
                for u in range(unroll):
                    v = raw_v[r, pl.ds((i * unroll + u) * nl, nl)]
                    s = s + v
                    s2 = s2 + v * v
                return s, s2

            zero = jnp.zeros((nl,), jnp.float32)
            s, s2 = lax.fori_loop(0, nch // unroll, stats, (zero, zero))
            tot = s[0]
            tot2 = s2[0]
            for j in range(1, nl):
                tot = tot + s[j]
                tot2 = tot2 + s2[j]
            mean = tot * (1.0 / hidden)
            var = tot2 * (1.0 / hidden) - mean * mean
            a = var + EPS
            y = lax.bitcast_convert_type(
                0x5F3759DF - lax.shift_right_logical(
                    lax.bitcast_convert_type(a, jnp.int32), 1),
                jnp.float32,
            )
            for _ in range(3):
                y = y * (1.5 - 0.5 * a * y * y)
            rstd = y

            def norm(i, carry):
                for u in range(unroll):
                    o = (i * unroll + u) * nl
                    v = raw_v[r, pl.ds(o, nl)]
                    nv = (v - mean) * rstd * w_v[pl.ds(o, nl)] + b_v[pl.ds(o, nl)]
                    for other in range(rows):
                        tab_v[(r * rows + other) * 2, pl.ds(o, nl)] = nv
                        tab_v[(other * rows + r) * 2 + 1, pl.ds(o, nl)] = nv
                return carry

            lax.fori_loop(0, nch // unroll, norm, 0)

        def group(g, carry):
            iv = idx_v[pl.ds(g * nl, nl)]
            for p in range(nl // 2):
                t = g * nl + 2 * p
                slot = (iv[2 * p] * rows + iv[2 * p + 1]) * 2
                pltpu.async_copy(
                    tab_v.at[pl.ds(slot, 2)],
                    out_hbm.at[pl.ds(base + t, 2)],
                    osem,
                )
            @pl.when(g > 0)
            def _():
                pltpu.make_async_copy(
                    out_hbm.at[pl.ds(base, nl)], drain_v, osem
                ).wait()
            return carry

        lax.fori_loop(0, ngroups, group, 0)

        pltpu.make_async_copy(out_hbm.at[pl.ds(base, nl)], drain_v, osem).wait()

    return sc_gather


def kernel(type_token, table, ln_weight, ln_bias):
    b, s = type_token.shape
    rows, hidden = table.shape
    tokens = b * s
    idx = type_token.reshape(tokens).astype(jnp.int32)
    out = _make_sc_gather(tokens, hidden, rows)(table, ln_weight, ln_bias, idx)
    return out.reshape(b, s, hidden)


# speedup vs baseline: 1.1160x; 1.1160x over previous
import functools

import jax
import jax.numpy as jnp
from jax import lax
from jax.experimental import pallas as pl
from jax.experimental.pallas import tpu as pltpu

EPS = 1e-5


def _ln_table_body(table_ref, w_ref, b_ref, out_ref):
    t = table_ref[...]
    mean = jnp.mean(t, axis=-1, keepdims=True)
    var = jnp.mean(jnp.square(t - mean), axis=-1, keepdims=True)
    out_ref[...] = (t - mean) * lax.rsqrt(var + EPS) * w_ref[...] + b_ref[...]


def _normalize_table(table, ln_weight, ln_bias):
    rows, hidden = table.shape
    return pl.pallas_call(
        _ln_table_body,
        out_shape=jax.ShapeDtypeStruct((rows, hidden), jnp.float32),
    )(table, ln_weight.reshape(1, hidden), ln_bias.reshape(1, hidden))


def _select_body(ids_ref, tab_ref, out_ref):
    ids = ids_ref[0]
    m0 = ids == 0
    m1 = ids == 1
    r0 = tab_ref[0, :][None, :]
    r1 = tab_ref[1, :][None, :]
    r2 = tab_ref[2, :][None, :]
    out_ref[...] = jnp.where(m0, r0, jnp.where(m1, r1, r2))


def _tc_select(normed, idx, tokens, hidden, tb):
    nblk = tokens // tb
    rows = normed.shape[0]
    return pl.pallas_call(
        _select_body,
        grid=(nblk,),
        in_specs=[
            pl.BlockSpec((1, tb, 1), lambda i: (i, 0, 0)),
            pl.BlockSpec((rows, hidden), lambda i: (0, 0)),
        ],
        out_specs=pl.BlockSpec((tb, hidden), lambda i: (i, 0)),
        out_shape=jax.ShapeDtypeStruct((tokens, hidden), jnp.float32),
    )(idx.reshape(nblk, tb, 1), normed)


def kernel(type_token, table, ln_weight, ln_bias):
    b, s = type_token.shape
    rows, hidden = table.shape
    tokens = b * s
    normed = _normalize_table(table, ln_weight, ln_bias)
    idx = type_token.reshape(tokens).astype(jnp.int32)
    out = _tc_select(normed, idx, tokens, hidden, tb=512)
    return out.reshape(b, s, hidden)
